# baseline (speedup 1.0000x reference)
import functools

import jax
import jax.numpy as jnp
from jax import lax
from jax.experimental import pallas as pl
from jax.experimental.pallas import tpu as pltpu


def _tin_kernel(x_ref, o_ref):
    o_ref[...] = x_ref[...].reshape(o_ref.shape)


def _gcn_kernel(nodes, bs, n_edges,
                t_ref, idx_ref, ea_ref, wms_ref, wme_ref, bm_ref, ws_ref,
                wa_ref, bo_ref, out_ref, y_scr, c_scr, agg_scr, cnt_scr):
    f = t_ref.shape[1]

    def y_body(i, _):
        blk = t_ref[pl.ds(i * bs, bs), :]
        y_scr[pl.ds(i * bs, bs), :] = jnp.dot(
            blk, wms_ref[...], preferred_element_type=jnp.float32)
        return 0
    lax.fori_loop(0, nodes, y_body, 0)

    e_chunk = 200
    def c_body(i, _):
        blk = ea_ref[pl.ds(i * e_chunk, e_chunk), :]
        c_scr[pl.ds(i * e_chunk, e_chunk), :] = (
            jnp.dot(blk, wme_ref[...], preferred_element_type=jnp.float32)
            + bm_ref[...])
        return 0
    lax.fori_loop(0, n_edges // e_chunk, c_body, 0)

    def z_body(i, _):
        agg_scr[pl.ds(i * bs, bs), :] = jnp.zeros((bs, f), jnp.float32)
        cnt_scr[i] = 0.0
        return 0
    lax.fori_loop(0, nodes, z_body, 0)

    def e_body(e, _):
        s = idx_ref[0, e]
        d = idx_ref[1, e]
        msg = jnp.maximum(
            y_scr[pl.ds(s * bs, bs), :] + c_scr[pl.ds(e, 1), :], 0.0)
        agg_scr[pl.ds(d * bs, bs), :] += msg
        cnt_scr[d] += 1.0
        return 0
    lax.fori_loop(0, n_edges, e_body, 0)

    def o_body(i, _):
        inv = 1.0 / jnp.maximum(cnt_scr[i], 1.0)
        tblk = t_ref[pl.ds(i * bs, bs), :]
        ablk = agg_scr[pl.ds(i * bs, bs), :] * inv
        h = (jnp.dot(tblk, ws_ref[...], preferred_element_type=jnp.float32)
             + jnp.dot(ablk, wa_ref[...], preferred_element_type=jnp.float32)
             + bo_ref[...])
        out_ref[pl.ds(i * bs, bs), :] = jnp.maximum(h, 0.0)
        return 0
    lax.fori_loop(0, nodes, o_body, 0)


def kernel(X, edge_index, edge_attr, W_msg_src, W_msg_edge, b_msg, W_self,
           W_agg, b_out):
    b, n, s, f_in = X.shape
    bs = b * s
    e = edge_index.shape[1]
    f_out = W_msg_src.shape[1]

    t4d = pl.pallas_call(
        _tin_kernel,
        grid=(n,),
        in_specs=[pl.BlockSpec((b, 1, s, f_in), lambda i: (0, i, 0, 0))],
        out_specs=pl.BlockSpec((1, b, s, f_in), lambda i: (i, 0, 0, 0)),
        out_shape=jax.ShapeDtypeStruct((n, b, s, f_in), jnp.float32),
    )(X)
    t2d = t4d.reshape(n * bs, f_in)

    bm2d = b_msg.reshape(1, f_out)
    bo2d = b_out.reshape(1, f_out)

    out2d = pl.pallas_call(
        functools.partial(_gcn_kernel, n, bs, e),
        out_shape=jax.ShapeDtypeStruct((n * bs, f_out), jnp.float32),
        in_specs=[
            pl.BlockSpec(memory_space=pltpu.VMEM),
            pl.BlockSpec(memory_space=pltpu.SMEM),
            pl.BlockSpec(memory_space=pltpu.VMEM),
            pl.BlockSpec(memory_space=pltpu.VMEM),
            pl.BlockSpec(memory_space=pltpu.VMEM),
            pl.BlockSpec(memory_space=pltpu.VMEM),
            pl.BlockSpec(memory_space=pltpu.VMEM),
            pl.BlockSpec(memory_space=pltpu.VMEM),
            pl.BlockSpec(memory_space=pltpu.VMEM),
        ],
        out_specs=pl.BlockSpec(memory_space=pltpu.VMEM),
        scratch_shapes=[
            pltpu.VMEM((n * bs, f_out), jnp.float32),
            pltpu.VMEM((e, f_out), jnp.float32),
            pltpu.VMEM((n * bs, f_out), jnp.float32),
            pltpu.SMEM((n,), jnp.float32),
        ],
    )(t2d, edge_index, edge_attr, W_msg_src, W_msg_edge, bm2d, W_self,
      W_agg, bo2d)

    out4d = out2d.reshape(n, b, s, f_out)
    return pl.pallas_call(
        _tin_kernel,
        grid=(n,),
        in_specs=[pl.BlockSpec((1, b, s, f_out), lambda i: (i, 0, 0, 0))],
        out_specs=pl.BlockSpec((b, 1, s, f_out), lambda i: (0, i, 0, 0)),
        out_shape=jax.ShapeDtypeStruct((b, n, s, f_out), jnp.float32),
    )(out4d)


# speedup vs baseline: 2.0675x; 2.0675x over previous
import functools

import jax
import jax.numpy as jnp
from jax import lax
from jax.experimental import pallas as pl
from jax.experimental.pallas import tpu as pltpu


def _tin_kernel(x_ref, o_ref):
    o_ref[...] = x_ref[...].reshape(o_ref.shape)


def _gcn_kernel(nodes, bs, n_edges,
                t_ref, idx_ref, ea_ref, wms_ref, wme_ref, bm_ref, ws_ref,
                wa_ref, bo_ref, out_ref, y_scr, c_scr, agg_scr, cnt_scr):
    f = t_ref.shape[1]

    def y_body(i, _):
        blk = t_ref[pl.ds(i * bs, bs), :]
        y_scr[pl.ds(i * bs, bs), :] = jnp.dot(
            blk, wms_ref[...], preferred_element_type=jnp.float32)
        return 0
    lax.fori_loop(0, nodes, y_body, 0)

    e_chunk = 200
    def c_body(i, _):
        blk = ea_ref[pl.ds(i * e_chunk, e_chunk), :]
        c_scr[pl.ds(i * e_chunk, e_chunk), :] = (
            jnp.dot(blk, wme_ref[...], preferred_element_type=jnp.float32)
            + bm_ref[...])
        return 0
    lax.fori_loop(0, n_edges // e_chunk, c_body, 0)

    def z_body(i, _):
        agg_scr[pl.ds(i * bs, bs), :] = jnp.zeros((bs, f), jnp.float32)
        cnt_scr[i] = 0.0
        return 0
    lax.fori_loop(0, nodes, z_body, 0)

    def e_body(e, _):
        s = idx_ref[0, e]
        d = idx_ref[1, e]
        msg = jnp.maximum(
            y_scr[pl.ds(s * bs, bs), :] + c_scr[pl.ds(e, 1), :], 0.0)
        agg_scr[pl.ds(d * bs, bs), :] += msg
        cnt_scr[d] += 1.0
        return 0
    lax.fori_loop(0, n_edges, e_body, 0)

    def o_body(i, _):
        inv = 1.0 / jnp.maximum(cnt_scr[i], 1.0)
        tblk = t_ref[pl.ds(i * bs, bs), :]
        ablk = agg_scr[pl.ds(i * bs, bs), :] * inv
        h = (jnp.dot(tblk, ws_ref[...], preferred_element_type=jnp.float32)
             + jnp.dot(ablk, wa_ref[...], preferred_element_type=jnp.float32)
             + bo_ref[...])
        out_ref[pl.ds(i * bs, bs), :] = jnp.maximum(h, 0.0)
        return 0
    lax.fori_loop(0, nodes, o_body, 0)


def kernel(X, edge_index, edge_attr, W_msg_src, W_msg_edge, b_msg, W_self,
           W_agg, b_out):
    b, n, s, f_in = X.shape
    bs = b * s
    e = edge_index.shape[1]
    f_out = W_msg_src.shape[1]

    t4d = pl.pallas_call(
        _tin_kernel,
        grid=(b,),
        in_specs=[pl.BlockSpec((1, n, s, f_in), lambda i: (i, 0, 0, 0))],
        out_specs=pl.BlockSpec((n, 1, s, f_in), lambda i: (0, i, 0, 0)),
        out_shape=jax.ShapeDtypeStruct((n, b, s, f_in), jnp.float32),
    )(X)
    t2d = t4d.reshape(n * bs, f_in)

    bm2d = b_msg.reshape(1, f_out)
    bo2d = b_out.reshape(1, f_out)

    out2d = pl.pallas_call(
        functools.partial(_gcn_kernel, n, bs, e),
        out_shape=jax.ShapeDtypeStruct((n * bs, f_out), jnp.float32),
        in_specs=[
            pl.BlockSpec(memory_space=pltpu.VMEM),
            pl.BlockSpec(memory_space=pltpu.SMEM),
            pl.BlockSpec(memory_space=pltpu.VMEM),
            pl.BlockSpec(memory_space=pltpu.VMEM),
            pl.BlockSpec(memory_space=pltpu.VMEM),
            pl.BlockSpec(memory_space=pltpu.VMEM),
            pl.BlockSpec(memory_space=pltpu.VMEM),
            pl.BlockSpec(memory_space=pltpu.VMEM),
            pl.BlockSpec(memory_space=pltpu.VMEM),
        ],
        out_specs=pl.BlockSpec(memory_space=pltpu.VMEM),
        scratch_shapes=[
            pltpu.VMEM((n * bs, f_out), jnp.float32),
            pltpu.VMEM((e, f_out), jnp.float32),
            pltpu.VMEM((n * bs, f_out), jnp.float32),
            pltpu.SMEM((n,), jnp.float32),
        ],
    )(t2d, edge_index, edge_attr, W_msg_src, W_msg_edge, bm2d, W_self,
      W_agg, bo2d)

    out4d = out2d.reshape(n, b, s, f_out)
    return pl.pallas_call(
        _tin_kernel,
        grid=(b,),
        in_specs=[pl.BlockSpec((n, 1, s, f_out), lambda i: (0, i, 0, 0))],
        out_specs=pl.BlockSpec((1, n, s, f_out), lambda i: (i, 0, 0, 0)),
        out_shape=jax.ShapeDtypeStruct((b, n, s, f_out), jnp.float32),
    )(out4d)


# speedup vs baseline: 4.1176x; 1.9916x over previous
import functools

import jax
import jax.numpy as jnp
from jax import lax
from jax.experimental import pallas as pl
from jax.experimental.pallas import tpu as pltpu


def _gcn_kernel(nodes, nb, ns, n_edges,
                x_ref, idx_ref, ea_ref, wms_ref, wme_ref, bm_ref, ws_ref,
                wa_ref, bo_ref, out_ref, y_scr, c_scr, agg_scr, cnt_scr):
    f = x_ref.shape[2]

    def y_body(i, _):
        y_scr[i] = jnp.dot(x_ref[i], wms_ref[...],
                           preferred_element_type=jnp.float32)
        return 0
    lax.fori_loop(0, nb, y_body, 0)

    e_chunk = 200
    def c_body(i, _):
        blk = ea_ref[pl.ds(i * e_chunk, e_chunk), :]
        c_scr[pl.ds(i * e_chunk, e_chunk), :] = (
            jnp.dot(blk, wme_ref[...], preferred_element_type=jnp.float32)
            + bm_ref[...])
        return 0
    lax.fori_loop(0, n_edges // e_chunk, c_body, 0)

    def z_body(i, _):
        agg_scr[i] = jnp.zeros(agg_scr.shape[1:], jnp.float32)
        return 0
    lax.fori_loop(0, nb, z_body, 0)

    def zc_body(i, _):
        cnt_scr[i] = 0.0
        return 0
    lax.fori_loop(0, nodes, zc_body, 0)

    def e_body(e, _):
        s = idx_ref[0, e]
        d = idx_ref[1, e]
        msg = jnp.maximum(
            y_scr[:, pl.ds(s * ns, ns), :] + c_scr[pl.ds(e, 1), :], 0.0)
        agg_scr[:, pl.ds(d * ns, ns), :] += msg
        cnt_scr[d] += 1.0
        return 0
    lax.fori_loop(0, n_edges, e_body, 0)

    def m_body(d, _):
        inv = 1.0 / jnp.maximum(cnt_scr[d], 1.0)
        agg_scr[:, pl.ds(d * ns, ns), :] *= inv
        return 0
    lax.fori_loop(0, nodes, m_body, 0)

    def o_body(i, _):
        h = (jnp.dot(x_ref[i], ws_ref[...], preferred_element_type=jnp.float32)
             + jnp.dot(agg_scr[i], wa_ref[...],
                       preferred_element_type=jnp.float32)
             + bo_ref[...])
        out_ref[i] = jnp.maximum(h, 0.0)
        return 0
    lax.fori_loop(0, nb, o_body, 0)


def kernel(X, edge_index, edge_attr, W_msg_src, W_msg_edge, b_msg, W_self,
           W_agg, b_out):
    b, n, s, f_in = X.shape
    e = edge_index.shape[1]
    f_out = W_msg_src.shape[1]

    x3 = X.reshape(b, n * s, f_in)
    bm2d = b_msg.reshape(1, f_out)
    bo2d = b_out.reshape(1, f_out)

    out3 = pl.pallas_call(
        functools.partial(_gcn_kernel, n, b, s, e),
        out_shape=jax.ShapeDtypeStruct((b, n * s, f_out), jnp.float32),
        in_specs=[
            pl.BlockSpec(memory_space=pltpu.VMEM),
            pl.BlockSpec(memory_space=pltpu.SMEM),
            pl.BlockSpec(memory_space=pltpu.VMEM),
            pl.BlockSpec(memory_space=pltpu.VMEM),
            pl.BlockSpec(memory_space=pltpu.VMEM),
            pl.BlockSpec(memory_space=pltpu.VMEM),
            pl.BlockSpec(memory_space=pltpu.VMEM),
            pl.BlockSpec(memory_space=pltpu.VMEM),
            pl.BlockSpec(memory_space=pltpu.VMEM),
        ],
        out_specs=pl.BlockSpec(memory_space=pltpu.VMEM),
        scratch_shapes=[
            pltpu.VMEM((b, n * s, f_out), jnp.float32),
            pltpu.VMEM((e, f_out), jnp.float32),
            pltpu.VMEM((b, n * s, f_out), jnp.float32),
            pltpu.SMEM((n,), jnp.float32),
        ],
    )(x3, edge_index, edge_attr, W_msg_src, W_msg_edge, bm2d, W_self,
      W_agg, bo2d)

    return out3.reshape(b, n, s, f_out)


# speedup vs baseline: 4.4482x; 1.0803x over previous
import functools

import jax
import jax.numpy as jnp
from jax import lax
from jax.experimental import pallas as pl
from jax.experimental.pallas import tpu as pltpu


def _gcn_kernel(nodes, nb, ns, n_edges,
                x_ref, idx_ref, ea_ref, wms_ref, wme_ref, bm_ref, ws_ref,
                wa_ref, bo_ref, out_ref, y_scr, c_scr, agg_scr, cnt_scr):
    f = x_ref.shape[2]

    def y_body(i, _):
        y_scr[i] = jnp.dot(x_ref[i], wms_ref[...],
                           preferred_element_type=jnp.float32)
        return 0
    lax.fori_loop(0, nb, y_body, 0)

    e_chunk = 200
    def c_body(i, _):
        blk = ea_ref[pl.ds(i * e_chunk, e_chunk), :]
        c_scr[pl.ds(i * e_chunk, e_chunk), :] = (
            jnp.dot(blk, wme_ref[...], preferred_element_type=jnp.float32)
            + bm_ref[...])
        return 0
    lax.fori_loop(0, n_edges // e_chunk, c_body, 0)

    def z_body(i, _):
        agg_scr[i] = jnp.zeros(agg_scr.shape[1:], jnp.float32)
        out_ref[i] = jnp.zeros(agg_scr.shape[1:], jnp.float32)
        return 0
    lax.fori_loop(0, nb, z_body, 0)

    def zc_body(i, _):
        cnt_scr[i] = 0.0
        return 0
    lax.fori_loop(0, nodes, zc_body, 0)

    def e_body(i, _):
        e0 = i * 2
        e1 = e0 + 1
        s0 = idx_ref[0, e0]
        d0 = idx_ref[1, e0]
        s1 = idx_ref[0, e1]
        d1 = idx_ref[1, e1]
        msg0 = jnp.maximum(
            y_scr[:, pl.ds(s0 * ns, ns), :] + c_scr[pl.ds(e0, 1), :], 0.0)
        msg1 = jnp.maximum(
            y_scr[:, pl.ds(s1 * ns, ns), :] + c_scr[pl.ds(e1, 1), :], 0.0)
        agg_scr[:, pl.ds(d0 * ns, ns), :] += msg0
        out_ref[:, pl.ds(d1 * ns, ns), :] += msg1
        cnt_scr[d0] += 1.0
        cnt_scr[d1] += 1.0
        return 0
    lax.fori_loop(0, n_edges // 2, e_body, 0)

    def m_body(d, _):
        inv = 1.0 / jnp.maximum(cnt_scr[d], 1.0)
        agg_scr[:, pl.ds(d * ns, ns), :] = (
            agg_scr[:, pl.ds(d * ns, ns), :]
            + out_ref[:, pl.ds(d * ns, ns), :]) * inv
        return 0
    lax.fori_loop(0, nodes, m_body, 0)

    def o_body(i, _):
        h = (jnp.dot(x_ref[i], ws_ref[...], preferred_element_type=jnp.float32)
             + jnp.dot(agg_scr[i], wa_ref[...],
                       preferred_element_type=jnp.float32)
             + bo_ref[...])
        out_ref[i] = jnp.maximum(h, 0.0)
        return 0
    lax.fori_loop(0, nb, o_body, 0)


def kernel(X, edge_index, edge_attr, W_msg_src, W_msg_edge, b_msg, W_self,
           W_agg, b_out):
    b, n, s, f_in = X.shape
    e = edge_index.shape[1]
    f_out = W_msg_src.shape[1]

    x3 = X.reshape(b, n * s, f_in)
    bm2d = b_msg.reshape(1, f_out)
    bo2d = b_out.reshape(1, f_out)

    out3 = pl.pallas_call(
        functools.partial(_gcn_kernel, n, b, s, e),
        out_shape=jax.ShapeDtypeStruct((b, n * s, f_out), jnp.float32),
        in_specs=[
            pl.BlockSpec(memory_space=pltpu.VMEM),
            pl.BlockSpec(memory_space=pltpu.SMEM),
            pl.BlockSpec(memory_space=pltpu.VMEM),
            pl.BlockSpec(memory_space=pltpu.VMEM),
            pl.BlockSpec(memory_space=pltpu.VMEM),
            pl.BlockSpec(memory_space=pltpu.VMEM),
            pl.BlockSpec(memory_space=pltpu.VMEM),
            pl.BlockSpec(memory_space=pltpu.VMEM),
            pl.BlockSpec(memory_space=pltpu.VMEM),
        ],
        out_specs=pl.BlockSpec(memory_space=pltpu.VMEM),
        scratch_shapes=[
            pltpu.VMEM((b, n * s, f_out), jnp.float32),
            pltpu.VMEM((e, f_out), jnp.float32),
            pltpu.VMEM((b, n * s, f_out), jnp.float32),
            pltpu.SMEM((n,), jnp.float32),
        ],
    )(x3, edge_index, edge_attr, W_msg_src, W_msg_edge, bm2d, W_self,
      W_agg, bo2d)

    return out3.reshape(b, n, s, f_out)
